# baseline (speedup 1.0000x reference)
import functools

import jax
import jax.numpy as jnp
from jax import lax
from jax.experimental import pallas as pl
from jax.experimental.pallas import tpu as pltpu
from jax.experimental.pallas import tpu_sc as plsc

_NC = 2
_NS = 16
_NW = _NC * _NS
_L = 16
_K = 128


def _sc_mesh():
    return plsc.VectorSubcoreMesh(core_axis_name="c", subcore_axis_name="s",
                                  num_cores=_NC, num_subcores=_NS)


def _fill_zeros(ref, rows, width):
    zv = jnp.zeros((_L,), jnp.float32)

    def row(r, carry):
        for k in range(width // _L):
            ref[r, pl.ds(k * _L, _L)] = zv
        return carry

    lax.fori_loop(0, rows, row, 0)


@functools.partial(jax.jit, static_argnums=(1, 2))
def _bincount(dst2, NP, EP):
    CW = EP // _K // _NW
    RT = NP // _NS

    def body(dst_hbm, out_hbm, dst_v, ones_v, zbuf, acc):
        cid = lax.axis_index("c")
        sid = lax.axis_index("s")
        wid = sid * _NC + cid

        ov = jnp.ones((_L,), jnp.float32)

        def orow(r, carry):
            ones_v[r, pl.ds(0, _L)] = ov
            return carry

        lax.fori_loop(0, _K, orow, 0)
        _fill_zeros(zbuf, RT, _L)

        pltpu.sync_copy(dst_hbm.at[pl.ds(wid * CW, CW)], dst_v)
        pltpu.sync_copy(zbuf, acc.at[pl.ds(sid * RT, RT)])
        plsc.subcore_barrier()

        def step(j, carry):
            pltpu.sync_copy(ones_v, acc.at[dst_v.at[j]], add=True)
            return carry

        lax.fori_loop(0, CW, step, 0)
        plsc.subcore_barrier()
        pltpu.sync_copy(acc.at[pl.ds(sid * RT, RT)],
                        out_hbm.at[cid, pl.ds(sid * RT, RT)])

    f = pl.kernel(
        body,
        out_type=jax.ShapeDtypeStruct((_NC, NP, _L), jnp.float32),
        mesh=_sc_mesh(),
        compiler_params=pltpu.CompilerParams(use_tc_tiling_on_sc=False,
                                             needs_layout_passes=False),
        scratch_types=[
            pltpu.VMEM((CW, _K), jnp.int32),
            pltpu.VMEM((_K, _L), jnp.float32),
            pltpu.VMEM((RT, _L), jnp.float32),
            pltpu.VMEM_SHARED((NP, _L), jnp.float32),
        ],
    )
    return f(dst2)


@functools.partial(jax.jit, static_argnums=(2, 3, 4, 5))
def _bucketize(src2, dst2, RR, MUL, SBH, EP):
    SB = 2 * SBH
    CW = EP // _K // _NW

    def body(src_hbm, dst_hbm, bpck_hbm, cnt_hbm,
             src_v, dst_v, pos_v, pck_v, row_v, cnt_sh, allcnt_v, ctr_sm):
        cid = lax.axis_index("c")
        sid = lax.axis_index("s")
        wid = sid * _NC + cid
        ila = lax.iota(jnp.int32, _L)

        pltpu.sync_copy(src_hbm.at[pl.ds(wid * CW, CW)], src_v)
        pltpu.sync_copy(dst_hbm.at[pl.ds(wid * CW, CW)], dst_v)

        def count_chunk(j, carry):
            lo, hi = carry
            for v in range(_K // _L):
                d = dst_v[j, pl.ds(v * _L, _L)]
                bv = (d * MUL) >> 22
                for b in range(_L):
                    pc = plsc.all_reduce_population_count(bv == b)
                    lo = lo + jnp.where(ila == b, pc, 0)
                for b in range(_L, 2 * _L):
                    pc = plsc.all_reduce_population_count(bv == b)
                    hi = hi + jnp.where(ila == (b - _L), pc, 0)
            return lo, hi

        z16 = jnp.zeros((_L,), jnp.int32)
        lo, hi = lax.fori_loop(0, CW, count_chunk, (z16, z16))
        row_v[0, pl.ds(0, _L)] = lo
        row_v[0, pl.ds(_L, _L)] = hi
        pltpu.sync_copy(row_v, cnt_sh.at[pl.ds(sid, 1)])
        plsc.subcore_barrier()

        pltpu.sync_copy(cnt_sh, allcnt_v)

        def presum(w, carry):
            lo_c, hi_c = carry
            sel = (w < sid).astype(jnp.int32)
            lo_c = lo_c + sel * allcnt_v[w, pl.ds(0, _L)]
            hi_c = hi_c + sel * allcnt_v[w, pl.ds(_L, _L)]
            return lo_c, hi_c

        plo, phi = lax.fori_loop(0, _NS, presum, (z16, z16))
        for b in range(32):
            k, l = divmod(b, _L)
            pre = (plo if k == 0 else phi)[l]
            ctr_sm[b] = b * SB + cid * SBH + pre

        @pl.when(sid == 0)
        def _():
            def totsum(w, carry):
                tl, th = carry
                tl = tl + allcnt_v[w, pl.ds(0, _L)]
                th = th + allcnt_v[w, pl.ds(_L, _L)]
                return tl, th

            tlo, thi = lax.fori_loop(0, _NS, totsum, (z16, z16))
            row_v[0, pl.ds(0, _L)] = tlo
            row_v[0, pl.ds(_L, _L)] = thi
            pltpu.sync_copy(row_v, cnt_hbm.at[pl.ds(cid, 1)])

        def scat_chunk(j, carry):
            for v in range(_K // _L):
                sl = pl.ds(v * _L, _L)
                d = dst_v[j, sl]
                s = src_v[j, sl]
                bv = (d * MUL) >> 22
                pos = jnp.zeros((_L,), jnp.int32)
                for b in range(32):
                    m = bv == b
                    pc = plsc.all_reduce_population_count(m)
                    base = ctr_sm[b]
                    ctr_sm[b] = base + pc[0]
                    rank = plsc.cumsum(m.astype(jnp.int32))
                    pos = jnp.where(m, base + rank - 1, pos)
                pos_v[0, sl] = pos
                pck_v[0, sl] = (s << 14) | d
            pltpu.sync_copy(pck_v.at[0], bpck_hbm.at[pos_v.at[0]])
            return carry

        lax.fori_loop(0, CW, scat_chunk, 0)

    f = pl.kernel(
        body,
        out_type=[jax.ShapeDtypeStruct((32 * SB,), jnp.int32),
                  jax.ShapeDtypeStruct((_NC, 32), jnp.int32)],
        mesh=_sc_mesh(),
        compiler_params=pltpu.CompilerParams(use_tc_tiling_on_sc=False,
                                             needs_layout_passes=False),
        scratch_types=[
            pltpu.VMEM((CW, _K), jnp.int32),
            pltpu.VMEM((CW, _K), jnp.int32),
            pltpu.VMEM((1, _K), jnp.int32),
            pltpu.VMEM((1, _K), jnp.int32),
            pltpu.VMEM((1, 32), jnp.int32),
            pltpu.VMEM_SHARED((_NS, 32), jnp.int32),
            pltpu.VMEM((_NS, 32), jnp.int32),
            pltpu.SMEM((32,), jnp.int32),
        ],
    )
    return f(src2, dst2)


@functools.partial(jax.jit, static_argnums=(4, 5, 6, 7))
def _segsum(tables, bpck, cnt, _unused, n_tables, RR, SBH, NPB):
    SB = 2 * SBH

    def body(*refs):
        tabs = refs[:n_tables]
        bpck_hbm, cnt_hbm, out_hbm = refs[n_tables:n_tables + 3]
        pbuf, sbuf, dloc, cntbuf, rows_v, acc, gsem = refs[n_tables + 3:]

        cid = lax.axis_index("c")
        sid = lax.axis_index("s")
        b = sid * _NC + cid

        pltpu.sync_copy(cnt_hbm, cntbuf.at[pl.ds(0, _NC), pl.ds(0, 32)])
        n0 = cntbuf[0, pl.ds(b, _L)][0]
        n1 = cntbuf[1, pl.ds(b, _L)][0]

        iotas = [lax.iota(jnp.int32, _L) + v * _L for v in range(_K // _L)]

        def process_run(tab, base, n):
            ncap = (n + _K - 1) // _K

            def chunk(j, carry):
                off = base + j * _K
                pltpu.sync_copy(bpck_hbm.at[pl.ds(off, _K)], pbuf.at[0])
                rem = n - j * _K
                for v in range(_K // _L):
                    sl = pl.ds(v * _L, _L)
                    keep = iotas[v] < rem
                    p = pbuf[0, sl]
                    sbuf[0, sl] = jnp.where(keep, p >> 14, 0)
                    dloc[0, sl] = jnp.where(keep, (p & 16383) - b * RR, RR)
                pltpu.async_copy(tab.at[sbuf.at[0]], rows_v, gsem).wait()

                def edge(e, carry2):
                    r = dloc[0, pl.ds(e, _L)][0]
                    for k in range(128 // _L):
                        sl = pl.ds(k * _L, _L)
                        plsc.addupdate(acc.at[r].at[sl], rows_v[e, sl])
                    return carry2

                lax.fori_loop(0, _K, edge, 0)
                return carry

            lax.fori_loop(0, ncap, chunk, 0)

        zv = jnp.zeros((_L,), jnp.float32)

        for t in range(n_tables):
            tab = tabs[t]

            def zrow(r, carry):
                for k in range(128 // _L):
                    acc[r, pl.ds(k * _L, _L)] = zv
                return carry

            lax.fori_loop(0, RR + 8, zrow, 0)
            process_run(tab, b * SB, n0)
            process_run(tab, b * SB + SBH, n1)
            pltpu.sync_copy(acc.at[pl.ds(0, RR)],
                            out_hbm.at[t, pl.ds(b * RR, RR)])

    f = pl.kernel(
        body,
        out_type=jax.ShapeDtypeStruct((n_tables, NPB, 128), jnp.float32),
        mesh=_sc_mesh(),
        compiler_params=pltpu.CompilerParams(use_tc_tiling_on_sc=False,
                                             needs_layout_passes=False),
        scratch_types=[
            pltpu.VMEM((1, _K), jnp.int32),
            pltpu.VMEM((1, _K), jnp.int32),
            pltpu.VMEM((1, _K + _L), jnp.int32),
            pltpu.VMEM((_NC, 48), jnp.int32),
            pltpu.VMEM((_K, 128), jnp.float32),
            pltpu.VMEM((RR + 8, 128), jnp.float32),
            pltpu.SemaphoreType.DMA,
        ],
    )
    return f(*tables, bpck, cnt)


def _row_block(N):
    for cand in (400, 500, 250, 200, 128, 100, 80, 50, 40, 25, 20, 16, 10, 8,
                 5, 4, 2, 1):
        if N % cand == 0:
            return cand
    return 1


def _disg0_call(degp, x, N, BR):
    F = x.shape[1]

    def body(degp_ref, x_ref, dis_ref, g0_ref):
        p = degp_ref[...]
        deg = 1.0 + p[0, :, 0:1] + p[1, :, 0:1]
        dis = lax.rsqrt(deg)
        dis_ref[...] = dis
        g0_ref[...] = x_ref[...] * dis

    return pl.pallas_call(
        body,
        grid=(N // BR,),
        in_specs=[
            pl.BlockSpec((_NC, BR, _L), lambda i: (0, i, 0)),
            pl.BlockSpec((BR, F), lambda i: (i, 0)),
        ],
        out_specs=[
            pl.BlockSpec((BR, 1), lambda i: (i, 0)),
            pl.BlockSpec((BR, F), lambda i: (i, 0)),
        ],
        out_shape=[
            jax.ShapeDtypeStruct((N, 1), jnp.float32),
            jax.ShapeDtypeStruct((N, F), jnp.float32),
        ],
    )(degp, x)


def _conv1_call(s0, g0, dis, W1, b1, W2, N, BR):
    F = g0.shape[1]
    H1 = W1.shape[1]
    H2 = W2.shape[1]
    NT1 = H2 // 128

    def body(s0_ref, g0_ref, dis_ref, W1_ref, b1_ref, W2_ref, *outs):
        dis = dis_ref[...]
        a = dis * (s0_ref[0] + g0_ref[...])
        h1 = jnp.maximum(
            jnp.dot(a, W1_ref[...], preferred_element_type=jnp.float32)
            + b1_ref[...], 0.0)
        g1 = dis * jnp.dot(h1, W2_ref[...], preferred_element_type=jnp.float32)
        for c in range(NT1):
            outs[c][...] = g1[:, c * 128:(c + 1) * 128]

    return pl.pallas_call(
        body,
        grid=(N // BR,),
        in_specs=[
            pl.BlockSpec((1, BR, F), lambda i: (0, i, 0)),
            pl.BlockSpec((BR, F), lambda i: (i, 0)),
            pl.BlockSpec((BR, 1), lambda i: (i, 0)),
            pl.BlockSpec((F, H1), lambda i: (0, 0)),
            pl.BlockSpec((1, H1), lambda i: (0, 0)),
            pl.BlockSpec((H1, H2), lambda i: (0, 0)),
        ],
        out_specs=[pl.BlockSpec((BR, 128), lambda i: (i, 0))] * NT1,
        out_shape=[jax.ShapeDtypeStruct((N, 128), jnp.float32)] * NT1,
    )(s0, g0, dis, W1, b1, W2)


def _conv2_call(s1, g1s, dis, b2, W3, N, BR):
    NT1 = len(g1s)
    H2 = NT1 * 128
    H3 = W3.shape[1]

    def body(s1_ref, *refs):
        g1_refs = refs[:NT1]
        dis_ref, b2_ref, W3_ref, g2_ref = refs[NT1:]
        dis = dis_ref[...]
        s = jnp.concatenate(
            [s1_ref[c] + g1_refs[c][...] for c in range(NT1)], axis=1)
        h2 = jnp.maximum(dis * s + b2_ref[...], 0.0)
        g2_ref[...] = dis * jnp.dot(h2, W3_ref[...],
                                    preferred_element_type=jnp.float32)

    return pl.pallas_call(
        body,
        grid=(N // BR,),
        in_specs=[pl.BlockSpec((NT1, BR, 128), lambda i: (0, i, 0))]
        + [pl.BlockSpec((BR, 128), lambda i: (i, 0))] * NT1
        + [
            pl.BlockSpec((BR, 1), lambda i: (i, 0)),
            pl.BlockSpec((1, H2), lambda i: (0, 0)),
            pl.BlockSpec((H2, H3), lambda i: (0, 0)),
        ],
        out_specs=pl.BlockSpec((BR, H3), lambda i: (i, 0)),
        out_shape=jax.ShapeDtypeStruct((N, H3), jnp.float32),
    )(s1, *g1s, dis, b2, W3)


def _conv3_call(s2, g2, dis, b3, Wfc, bfc, N, BR):
    H3 = g2.shape[1]
    C = Wfc.shape[1]

    def body(s2_ref, g2_ref, dis_ref, b3_ref, Wfc_ref, bfc_ref, out_ref):
        dis = dis_ref[...]
        h3 = jnp.maximum(
            dis * (s2_ref[0] + g2_ref[...]) + b3_ref[...], 0.0)
        out_ref[...] = (
            jnp.dot(h3, Wfc_ref[...], preferred_element_type=jnp.float32)
            + bfc_ref[...])

    return pl.pallas_call(
        body,
        grid=(N // BR,),
        in_specs=[
            pl.BlockSpec((1, BR, H3), lambda i: (0, i, 0)),
            pl.BlockSpec((BR, H3), lambda i: (i, 0)),
            pl.BlockSpec((BR, 1), lambda i: (i, 0)),
            pl.BlockSpec((1, H3), lambda i: (0, 0)),
            pl.BlockSpec((H3, C), lambda i: (0, 0)),
            pl.BlockSpec((1, C), lambda i: (0, 0)),
        ],
        out_specs=pl.BlockSpec((BR, C), lambda i: (i, 0)),
        out_shape=jax.ShapeDtypeStruct((N, C), jnp.float32),
    )(s2, g2, dis, b3, Wfc, bfc)


def kernel(x, edge_index, W1, b1, W2, b2, W3, b3, Wfc, bfc):
    N, F = x.shape
    E = edge_index.shape[1]

    grain = _NW * _K * 8
    EP = ((E + grain - 1) // grain) * grain
    RR = ((N + 1 + _NW - 1) // _NW + 7) // 8 * 8
    NPB = _NW * RR
    MUL = (1 << 22) // RR + 1
    SBH = EP // 2 + _K
    BR = _row_block(N)

    src = edge_index[0]
    dst = edge_index[1]
    if EP > E:
        pad = EP - E
        src = jnp.concatenate([src, jnp.zeros((pad,), jnp.int32)])
        dst = jnp.concatenate([dst, jnp.full((pad,), N, jnp.int32)])
    src2 = src.reshape(EP // _K, _K)
    dst2 = dst.reshape(EP // _K, _K)

    degp = _bincount(dst2, NPB, EP)
    dis, g0 = _disg0_call(degp, x, N, BR)

    bpck, cnt = _bucketize(src2, dst2, RR, MUL, SBH, EP)

    s0 = _segsum((g0,), bpck, cnt, 0, 1, RR, SBH, NPB)
    g1s = _conv1_call(s0, g0, dis, W1, b1.reshape(1, -1), W2, N, BR)

    s1 = _segsum(tuple(g1s), bpck, cnt, 0, len(g1s), RR, SBH, NPB)
    g2 = _conv2_call(s1, g1s, dis, b2.reshape(1, -1), W3, N, BR)

    s2 = _segsum((g2,), bpck, cnt, 0, 1, RR, SBH, NPB)
    out = _conv3_call(s2, g2, dis, b3.reshape(1, -1), Wfc,
                      bfc.reshape(1, -1), N, BR)
    return out


# speedup vs baseline: 1.6368x; 1.6368x over previous
import functools

import jax
import jax.numpy as jnp
from jax import lax
from jax.experimental import pallas as pl
from jax.experimental.pallas import tpu as pltpu
from jax.experimental.pallas import tpu_sc as plsc

_NC = 2
_NS = 16
_NW = _NC * _NS
_L = 16
_K = 128


def _sc_mesh():
    return plsc.VectorSubcoreMesh(core_axis_name="c", subcore_axis_name="s",
                                  num_cores=_NC, num_subcores=_NS)


def _fill_zeros(ref, rows, width):
    zv = jnp.zeros((_L,), jnp.float32)

    def row(r, carry):
        for k in range(width // _L):
            ref[r, pl.ds(k * _L, _L)] = zv
        return carry

    lax.fori_loop(0, rows, row, 0)


@functools.partial(jax.jit, static_argnums=(1, 2))
def _bincount(dst2, NP, EP):
    CW = EP // _K // _NW
    RT = NP // _NS

    def body(dst_hbm, out_hbm, dst_v, ones_v, zbuf, acc):
        cid = lax.axis_index("c")
        sid = lax.axis_index("s")
        wid = sid * _NC + cid

        ov = jnp.ones((_L,), jnp.float32)

        def orow(r, carry):
            ones_v[r, pl.ds(0, _L)] = ov
            return carry

        lax.fori_loop(0, _K, orow, 0)
        _fill_zeros(zbuf, RT, _L)

        pltpu.sync_copy(dst_hbm.at[pl.ds(wid * CW, CW)], dst_v)
        pltpu.sync_copy(zbuf, acc.at[pl.ds(sid * RT, RT)])
        plsc.subcore_barrier()

        def step(j, carry):
            pltpu.sync_copy(ones_v, acc.at[dst_v.at[j]], add=True)
            return carry

        lax.fori_loop(0, CW, step, 0)
        plsc.subcore_barrier()
        pltpu.sync_copy(acc.at[pl.ds(sid * RT, RT)],
                        out_hbm.at[cid, pl.ds(sid * RT, RT)])

    f = pl.kernel(
        body,
        out_type=jax.ShapeDtypeStruct((_NC, NP, _L), jnp.float32),
        mesh=_sc_mesh(),
        compiler_params=pltpu.CompilerParams(use_tc_tiling_on_sc=False,
                                             needs_layout_passes=False),
        scratch_types=[
            pltpu.VMEM((CW, _K), jnp.int32),
            pltpu.VMEM((_K, _L), jnp.float32),
            pltpu.VMEM((RT, _L), jnp.float32),
            pltpu.VMEM_SHARED((NP, _L), jnp.float32),
        ],
    )
    return f(dst2)


@functools.partial(jax.jit, static_argnums=(2, 3, 4, 5))
def _bucketize(src2, dst2, RR, MUL, SBH, EP):
    SB = 2 * SBH
    CW = EP // _K // _NW

    def body(src_hbm, dst_hbm, bpck_hbm, cnt_hbm,
             src_v, dst_v, pos_v, pck_v, row_v, cnt_sh, allcnt_v, ctr_sm):
        cid = lax.axis_index("c")
        sid = lax.axis_index("s")
        wid = sid * _NC + cid
        ila = lax.iota(jnp.int32, _L)

        pltpu.sync_copy(src_hbm.at[pl.ds(wid * CW, CW)], src_v)
        pltpu.sync_copy(dst_hbm.at[pl.ds(wid * CW, CW)], dst_v)

        def count_chunk(j, carry):
            lo, hi = carry
            for v in range(_K // _L):
                d = dst_v[j, pl.ds(v * _L, _L)]
                bv = (d * MUL) >> 22
                for b in range(_L):
                    pc = plsc.all_reduce_population_count(bv == b)
                    lo = lo + jnp.where(ila == b, pc, 0)
                for b in range(_L, 2 * _L):
                    pc = plsc.all_reduce_population_count(bv == b)
                    hi = hi + jnp.where(ila == (b - _L), pc, 0)
            return lo, hi

        z16 = jnp.zeros((_L,), jnp.int32)
        lo, hi = lax.fori_loop(0, CW, count_chunk, (z16, z16))
        row_v[0, pl.ds(0, _L)] = lo
        row_v[0, pl.ds(_L, _L)] = hi
        pltpu.sync_copy(row_v, cnt_sh.at[pl.ds(sid, 1)])
        plsc.subcore_barrier()

        pltpu.sync_copy(cnt_sh, allcnt_v)

        def presum(w, carry):
            lo_c, hi_c = carry
            sel = (w < sid).astype(jnp.int32)
            lo_c = lo_c + sel * allcnt_v[w, pl.ds(0, _L)]
            hi_c = hi_c + sel * allcnt_v[w, pl.ds(_L, _L)]
            return lo_c, hi_c

        plo, phi = lax.fori_loop(0, _NS, presum, (z16, z16))
        for b in range(32):
            k, l = divmod(b, _L)
            pre = (plo if k == 0 else phi)[l]
            ctr_sm[b] = b * SB + cid * SBH + pre

        @pl.when(sid == 0)
        def _():
            def totsum(w, carry):
                tl, th = carry
                tl = tl + allcnt_v[w, pl.ds(0, _L)]
                th = th + allcnt_v[w, pl.ds(_L, _L)]
                return tl, th

            tlo, thi = lax.fori_loop(0, _NS, totsum, (z16, z16))
            row_v[0, pl.ds(0, _L)] = tlo
            row_v[0, pl.ds(_L, _L)] = thi
            pltpu.sync_copy(row_v, cnt_hbm.at[pl.ds(cid, 1)])

        def scat_chunk(j, carry):
            for v in range(_K // _L):
                sl = pl.ds(v * _L, _L)
                d = dst_v[j, sl]
                s = src_v[j, sl]
                bv = (d * MUL) >> 22
                pos = jnp.zeros((_L,), jnp.int32)
                for b in range(32):
                    m = bv == b
                    pc = plsc.all_reduce_population_count(m)
                    base = ctr_sm[b]
                    ctr_sm[b] = base + pc[0]
                    rank = plsc.cumsum(m.astype(jnp.int32))
                    pos = jnp.where(m, base + rank - 1, pos)
                pos_v[0, sl] = pos
                pck_v[0, sl] = (s << 14) | d
            pltpu.sync_copy(pck_v.at[0], bpck_hbm.at[pos_v.at[0]])
            return carry

        lax.fori_loop(0, CW, scat_chunk, 0)

    f = pl.kernel(
        body,
        out_type=[jax.ShapeDtypeStruct((32 * SB,), jnp.int32),
                  jax.ShapeDtypeStruct((_NC, 32), jnp.int32)],
        mesh=_sc_mesh(),
        compiler_params=pltpu.CompilerParams(use_tc_tiling_on_sc=False,
                                             needs_layout_passes=False),
        scratch_types=[
            pltpu.VMEM((CW, _K), jnp.int32),
            pltpu.VMEM((CW, _K), jnp.int32),
            pltpu.VMEM((1, _K), jnp.int32),
            pltpu.VMEM((1, _K), jnp.int32),
            pltpu.VMEM((1, 32), jnp.int32),
            pltpu.VMEM_SHARED((_NS, 32), jnp.int32),
            pltpu.VMEM((_NS, 32), jnp.int32),
            pltpu.SMEM((32,), jnp.int32),
        ],
    )
    return f(src2, dst2)


@functools.partial(jax.jit, static_argnums=(4, 5, 6, 7))
def _segsum(tables, bpck, cnt, _unused, n_tables, RR, SBH, NPB):
    SB = 2 * SBH
    ACC_ROWS = NPB + 32 * 8

    def body(*refs):
        tabs = refs[:n_tables]
        bpck_hbm, cnt_hbm, out_hbm = refs[n_tables:n_tables + 3]
        pbuf, sbuf, dbuf, cntbuf, rows_v, zbuf, acc, psem0, psem1, \
            gsem0, gsem1 = refs[n_tables + 3:]
        psem = (psem0, psem1)
        gsem = (gsem0, gsem1)

        cid = lax.axis_index("c")
        sid = lax.axis_index("s")
        b = sid * _NC + cid
        dummy = NPB + b * 8

        pltpu.sync_copy(cnt_hbm, cntbuf.at[pl.ds(0, _NC), pl.ds(0, 32)])
        _fill_zeros(zbuf, 32, 128)
        n0 = cntbuf[0, pl.ds(b, _L)][0]
        n1 = cntbuf[1, pl.ds(b, _L)][0]

        iotas = [lax.iota(jnp.int32, _L) + v * _L for v in range(_K // _L)]

        def process_run(tab, base, n):
            ncap = (n + _K - 1) // _K

            def pload(j, r):
                return pltpu.make_async_copy(
                    bpck_hbm.at[pl.ds(base + j * _K, _K)], pbuf.at[r],
                    psem[r])

            def gath(r):
                return pltpu.make_async_copy(tab.at[sbuf.at[r]], rows_v.at[r],
                                             gsem[r])

            def unpack(j, r, n):
                rem = n - j * _K
                for v in range(_K // _L):
                    sl = pl.ds(v * _L, _L)
                    keep = iotas[v] < rem
                    p = pbuf[r, sl]
                    sbuf[r, sl] = jnp.where(keep, p >> 14, 0)
                    dbuf[r, sl] = jnp.where(keep, p & 16383, dummy)

            @pl.when(ncap > 0)
            def _():
                pload(0, 0).start()
                pload(0, 0).wait()
                unpack(0, 0, n)
                gath(0).start()

            @pl.when(ncap > 1)
            def _():
                pload(1, 1).start()

            @pl.loop(0, ncap, step=2)
            def _outer(j):
                for s in range(2):
                    jb = j + s
                    nb = (s + 1) % 2

                    @pl.when(jb + 1 < ncap)
                    def _():
                        pload(jb + 1, nb).wait()
                        unpack(jb + 1, nb, n)
                        gath(nb).start()

                    @pl.when(jb + 2 < ncap)
                    def _():
                        pload(jb + 2, s).start()

                    @pl.when(jb < ncap)
                    def _():
                        gath(s).wait()
                        pltpu.sync_copy(rows_v.at[s], acc.at[dbuf.at[s]],
                                        add=True)

            lax.fori_loop(0, 0, lambda i, c: c, 0)

        for t in range(n_tables):
            tab = tabs[t]
            for z in range(RR // 32):
                pltpu.sync_copy(zbuf, acc.at[pl.ds(b * RR + z * 32, 32)])
            pltpu.sync_copy(zbuf.at[pl.ds(0, 8)], acc.at[pl.ds(dummy, 8)])
            process_run(tab, b * SB, n0)
            process_run(tab, b * SB + SBH, n1)
            pltpu.sync_copy(acc.at[pl.ds(b * RR, RR)],
                            out_hbm.at[t, pl.ds(b * RR, RR)])

    f = pl.kernel(
        body,
        out_type=jax.ShapeDtypeStruct((n_tables, NPB, 128), jnp.float32),
        mesh=_sc_mesh(),
        compiler_params=pltpu.CompilerParams(use_tc_tiling_on_sc=False,
                                             needs_layout_passes=False),
        scratch_types=[
            pltpu.VMEM((2, _K), jnp.int32),
            pltpu.VMEM((2, _K), jnp.int32),
            pltpu.VMEM((2, _K), jnp.int32),
            pltpu.VMEM((_NC, 48), jnp.int32),
            pltpu.VMEM((2, _K, 128), jnp.float32),
            pltpu.VMEM((32, 128), jnp.float32),
            pltpu.VMEM_SHARED((ACC_ROWS, 128), jnp.float32),
            pltpu.SemaphoreType.DMA,
            pltpu.SemaphoreType.DMA,
            pltpu.SemaphoreType.DMA,
            pltpu.SemaphoreType.DMA,
        ],
    )
    return f(*tables, bpck, cnt)


def _row_block(N):
    for cand in (400, 500, 250, 200, 128, 100, 80, 50, 40, 25, 20, 16, 10, 8,
                 5, 4, 2, 1):
        if N % cand == 0:
            return cand
    return 1


def _disg0_call(degp, x, N, BR):
    F = x.shape[1]

    def body(degp_ref, x_ref, dis_ref, g0_ref):
        p = degp_ref[...]
        deg = 1.0 + p[0, :, 0:1] + p[1, :, 0:1]
        dis = lax.rsqrt(deg)
        dis_ref[...] = dis
        g0_ref[...] = x_ref[...] * dis

    return pl.pallas_call(
        body,
        grid=(N // BR,),
        in_specs=[
            pl.BlockSpec((_NC, BR, _L), lambda i: (0, i, 0)),
            pl.BlockSpec((BR, F), lambda i: (i, 0)),
        ],
        out_specs=[
            pl.BlockSpec((BR, 1), lambda i: (i, 0)),
            pl.BlockSpec((BR, F), lambda i: (i, 0)),
        ],
        out_shape=[
            jax.ShapeDtypeStruct((N, 1), jnp.float32),
            jax.ShapeDtypeStruct((N, F), jnp.float32),
        ],
    )(degp, x)


def _conv1_call(s0, g0, dis, W1, b1, W2, N, BR):
    F = g0.shape[1]
    H1 = W1.shape[1]
    H2 = W2.shape[1]
    NT1 = H2 // 128

    def body(s0_ref, g0_ref, dis_ref, W1_ref, b1_ref, W2_ref, *outs):
        dis = dis_ref[...]
        a = dis * (s0_ref[0] + g0_ref[...])
        h1 = jnp.maximum(
            jnp.dot(a, W1_ref[...], preferred_element_type=jnp.float32)
            + b1_ref[...], 0.0)
        g1 = dis * jnp.dot(h1, W2_ref[...], preferred_element_type=jnp.float32)
        for c in range(NT1):
            outs[c][...] = g1[:, c * 128:(c + 1) * 128]

    return pl.pallas_call(
        body,
        grid=(N // BR,),
        in_specs=[
            pl.BlockSpec((1, BR, F), lambda i: (0, i, 0)),
            pl.BlockSpec((BR, F), lambda i: (i, 0)),
            pl.BlockSpec((BR, 1), lambda i: (i, 0)),
            pl.BlockSpec((F, H1), lambda i: (0, 0)),
            pl.BlockSpec((1, H1), lambda i: (0, 0)),
            pl.BlockSpec((H1, H2), lambda i: (0, 0)),
        ],
        out_specs=[pl.BlockSpec((BR, 128), lambda i: (i, 0))] * NT1,
        out_shape=[jax.ShapeDtypeStruct((N, 128), jnp.float32)] * NT1,
    )(s0, g0, dis, W1, b1, W2)


def _conv2_call(s1, g1s, dis, b2, W3, N, BR):
    NT1 = len(g1s)
    H2 = NT1 * 128
    H3 = W3.shape[1]

    def body(s1_ref, *refs):
        g1_refs = refs[:NT1]
        dis_ref, b2_ref, W3_ref, g2_ref = refs[NT1:]
        dis = dis_ref[...]
        s = jnp.concatenate(
            [s1_ref[c] + g1_refs[c][...] for c in range(NT1)], axis=1)
        h2 = jnp.maximum(dis * s + b2_ref[...], 0.0)
        g2_ref[...] = dis * jnp.dot(h2, W3_ref[...],
                                    preferred_element_type=jnp.float32)

    return pl.pallas_call(
        body,
        grid=(N // BR,),
        in_specs=[pl.BlockSpec((NT1, BR, 128), lambda i: (0, i, 0))]
        + [pl.BlockSpec((BR, 128), lambda i: (i, 0))] * NT1
        + [
            pl.BlockSpec((BR, 1), lambda i: (i, 0)),
            pl.BlockSpec((1, H2), lambda i: (0, 0)),
            pl.BlockSpec((H2, H3), lambda i: (0, 0)),
        ],
        out_specs=pl.BlockSpec((BR, H3), lambda i: (i, 0)),
        out_shape=jax.ShapeDtypeStruct((N, H3), jnp.float32),
    )(s1, *g1s, dis, b2, W3)


def _conv3_call(s2, g2, dis, b3, Wfc, bfc, N, BR):
    H3 = g2.shape[1]
    C = Wfc.shape[1]

    def body(s2_ref, g2_ref, dis_ref, b3_ref, Wfc_ref, bfc_ref, out_ref):
        dis = dis_ref[...]
        h3 = jnp.maximum(
            dis * (s2_ref[0] + g2_ref[...]) + b3_ref[...], 0.0)
        out_ref[...] = (
            jnp.dot(h3, Wfc_ref[...], preferred_element_type=jnp.float32)
            + bfc_ref[...])

    return pl.pallas_call(
        body,
        grid=(N // BR,),
        in_specs=[
            pl.BlockSpec((1, BR, H3), lambda i: (0, i, 0)),
            pl.BlockSpec((BR, H3), lambda i: (i, 0)),
            pl.BlockSpec((BR, 1), lambda i: (i, 0)),
            pl.BlockSpec((1, H3), lambda i: (0, 0)),
            pl.BlockSpec((H3, C), lambda i: (0, 0)),
            pl.BlockSpec((1, C), lambda i: (0, 0)),
        ],
        out_specs=pl.BlockSpec((BR, C), lambda i: (i, 0)),
        out_shape=jax.ShapeDtypeStruct((N, C), jnp.float32),
    )(s2, g2, dis, b3, Wfc, bfc)


def kernel(x, edge_index, W1, b1, W2, b2, W3, b3, Wfc, bfc):
    N, F = x.shape
    E = edge_index.shape[1]

    grain = _NW * _K * 8
    EP = ((E + grain - 1) // grain) * grain
    RR = ((N + 1 + _NW - 1) // _NW + 7) // 8 * 8
    NPB = _NW * RR
    MUL = (1 << 22) // RR + 1
    SBH = EP // 2 + _K
    BR = _row_block(N)

    src = edge_index[0]
    dst = edge_index[1]
    if EP > E:
        pad = EP - E
        src = jnp.concatenate([src, jnp.zeros((pad,), jnp.int32)])
        dst = jnp.concatenate([dst, jnp.full((pad,), N, jnp.int32)])
    src2 = src.reshape(EP // _K, _K)
    dst2 = dst.reshape(EP // _K, _K)

    degp = _bincount(dst2, NPB, EP)
    dis, g0 = _disg0_call(degp, x, N, BR)

    bpck, cnt = _bucketize(src2, dst2, RR, MUL, SBH, EP)

    s0 = _segsum((g0,), bpck, cnt, 0, 1, RR, SBH, NPB)
    g1s = _conv1_call(s0, g0, dis, W1, b1.reshape(1, -1), W2, N, BR)

    s1 = _segsum(tuple(g1s), bpck, cnt, 0, len(g1s), RR, SBH, NPB)
    g2 = _conv2_call(s1, g1s, dis, b2.reshape(1, -1), W3, N, BR)

    s2 = _segsum((g2,), bpck, cnt, 0, 1, RR, SBH, NPB)
    out = _conv3_call(s2, g2, dis, b3.reshape(1, -1), Wfc,
                      bfc.reshape(1, -1), N, BR)
    return out


# speedup vs baseline: 2.2658x; 1.3843x over previous
import functools

import jax
import jax.numpy as jnp
from jax import lax
from jax.experimental import pallas as pl
from jax.experimental.pallas import tpu as pltpu
from jax.experimental.pallas import tpu_sc as plsc

_NC = 2
_NS = 16
_NW = _NC * _NS
_L = 16
_K = 128
_TC = 64


def _sc_mesh():
    return plsc.VectorSubcoreMesh(core_axis_name="c", subcore_axis_name="s",
                                  num_cores=_NC, num_subcores=_NS)


def _fill_zeros(ref, rows, width):
    zv = jnp.zeros((_L,), jnp.float32)

    def row(r, carry):
        for k in range(width // _L):
            ref[r, pl.ds(k * _L, _L)] = zv
        return carry

    lax.fori_loop(0, rows, row, 0)


@functools.partial(jax.jit, static_argnums=(1, 2))
def _bincount(dst2, NP, EP):
    CW = EP // _K // _NW
    RT = NP // _NS

    def body(dst_hbm, out_hbm, dst_v, ones_v, zbuf, acc):
        cid = lax.axis_index("c")
        sid = lax.axis_index("s")
        wid = sid * _NC + cid

        ov = jnp.ones((_L,), jnp.float32)

        def orow(r, carry):
            ones_v[r, pl.ds(0, _L)] = ov
            return carry

        lax.fori_loop(0, _K, orow, 0)
        _fill_zeros(zbuf, RT, _L)

        pltpu.sync_copy(dst_hbm.at[pl.ds(wid * CW, CW)], dst_v)
        pltpu.sync_copy(zbuf, acc.at[pl.ds(sid * RT, RT)])
        plsc.subcore_barrier()

        def step(j, carry):
            pltpu.sync_copy(ones_v, acc.at[dst_v.at[j]], add=True)
            return carry

        lax.fori_loop(0, CW, step, 0)
        plsc.subcore_barrier()
        pltpu.sync_copy(acc.at[pl.ds(sid * RT, RT)],
                        out_hbm.at[cid, pl.ds(sid * RT, RT)])

    f = pl.kernel(
        body,
        out_type=jax.ShapeDtypeStruct((_NC, NP, _L), jnp.float32),
        mesh=_sc_mesh(),
        compiler_params=pltpu.CompilerParams(use_tc_tiling_on_sc=False),
        scratch_types=[
            pltpu.VMEM((CW, _K), jnp.int32),
            pltpu.VMEM((_K, _L), jnp.float32),
            pltpu.VMEM((RT, _L), jnp.float32),
            pltpu.VMEM_SHARED((NP, _L), jnp.float32),
        ],
    )
    return f(dst2)


@functools.partial(jax.jit, static_argnums=(3, 4, 5, 6))
def _segsum(tables, src2, dst2, n_tables, NP, EP, SPLIT8):
    CH16 = EP // _K // _NS
    CW0 = CH16 * SPLIT8 // 8
    CW1 = CH16 - CW0
    C0T = _NS * CW0
    CW = max(CW0, CW1)
    RT = NP // _NS
    ZR = next(d for d in (32, 16, 8, RT) if RT % d == 0)

    NB = 4
    LD = 2

    def body(*refs):
        tabs = refs[:n_tables]
        src_hbm, dst_hbm, out_hbm = refs[n_tables:n_tables + 3]
        src_v, dst_v, rows_v, zbuf, acc, gsem = refs[n_tables + 3:]

        cid = lax.axis_index("c")
        sid = lax.axis_index("s")
        wid = sid * _NC + cid

        def gather(tab, c, b):
            return pltpu.make_async_copy(tab.at[src_v.at[c]], rows_v.at[b],
                                         gsem.at[b])

        _fill_zeros(zbuf, ZR, _TC)
        cw = jnp.where(cid == 0, CW0, CW1)

        @pl.when(cid == 0)
        def _():
            pltpu.sync_copy(src_hbm.at[pl.ds(sid * CW0, CW0)],
                            src_v.at[pl.ds(0, CW0)])
            pltpu.sync_copy(dst_hbm.at[pl.ds(sid * CW0, CW0)],
                            dst_v.at[pl.ds(0, CW0)])

        @pl.when(cid == 1)
        def _():
            pltpu.sync_copy(src_hbm.at[pl.ds(C0T + sid * CW1, CW1)],
                            src_v.at[pl.ds(0, CW1)])
            pltpu.sync_copy(dst_hbm.at[pl.ds(C0T + sid * CW1, CW1)],
                            dst_v.at[pl.ds(0, CW1)])

        for t in range(n_tables):
            tab = tabs[t]
            for c in range(LD):
                @pl.when(c < cw)
                def _(c=c):
                    gather(tab, c, c).start()
            for z in range(RT // ZR):
                pltpu.sync_copy(zbuf, acc.at[pl.ds(sid * RT + z * ZR, ZR)])
            plsc.subcore_barrier()

            @pl.loop(0, cw, step=NB)
            def _chunks(j):
                for i in range(NB):
                    jb = j + i
                    c = jb + LD
                    bg = (i + LD) % NB

                    @pl.when(c < cw)
                    def _():
                        gather(tab, c, bg).start()

                    gather(tab, jb, i).wait()
                    pltpu.sync_copy(rows_v.at[i], acc.at[dst_v.at[jb]],
                                    add=True)

            plsc.subcore_barrier()
            pltpu.sync_copy(acc.at[pl.ds(sid * RT, RT)],
                            out_hbm.at[cid, t, pl.ds(sid * RT, RT)])
            if t + 1 < n_tables:
                plsc.subcore_barrier()

    f = pl.kernel(
        body,
        out_type=jax.ShapeDtypeStruct((_NC, n_tables, NP, _TC), jnp.float32),
        mesh=_sc_mesh(),
        compiler_params=pltpu.CompilerParams(use_tc_tiling_on_sc=False),
        scratch_types=[
            pltpu.VMEM((CW, _K), jnp.int32),
            pltpu.VMEM((CW, _K), jnp.int32),
            pltpu.VMEM((NB, _K, _TC), jnp.float32),
            pltpu.VMEM((ZR, _TC), jnp.float32),
            pltpu.VMEM_SHARED((NP, _TC), jnp.float32),
            pltpu.SemaphoreType.DMA((NB,)),
        ],
    )
    return f(*tables, src2, dst2)


def _row_block(N):
    for cand in (400, 500, 250, 200, 128, 100, 80, 50, 40, 25, 20, 16, 10, 8, 5, 4, 2, 1):
        if N % cand == 0:
            return cand
    return 1


def _disg0_call(degp, x, N, BR):
    F = x.shape[1]
    NT = F // _TC

    def body(degp_ref, x_ref, dis_ref, *outs):
        p = degp_ref[...]
        deg = 1.0 + p[0, :, 0:1] + p[1, :, 0:1]
        dis = lax.rsqrt(deg)
        dis_ref[...] = dis
        g0 = x_ref[...] * dis
        for c in range(NT):
            outs[c][...] = g0[:, c * _TC:(c + 1) * _TC]

    return pl.pallas_call(
        body,
        grid=(N // BR,),
        in_specs=[
            pl.BlockSpec((_NC, BR, _L), lambda i: (0, i, 0)),
            pl.BlockSpec((BR, F), lambda i: (i, 0)),
        ],
        out_specs=[pl.BlockSpec((BR, 1), lambda i: (i, 0))]
        + [pl.BlockSpec((BR, _TC), lambda i: (i, 0))] * NT,
        out_shape=[jax.ShapeDtypeStruct((N, 1), jnp.float32)]
        + [jax.ShapeDtypeStruct((N, _TC), jnp.float32)] * NT,
    )(degp, x)


def _conv1_call(s0p, g0s, dis, W1, b1, W2, N, BR):
    NT0 = len(g0s)
    F = NT0 * _TC
    H1 = W1.shape[1]
    H2 = W2.shape[1]
    NT1 = H2 // _TC

    def body(s0p_ref, *refs):
        g0_refs = refs[:NT0]
        dis_ref, W1_ref, b1_ref, W2_ref = refs[NT0:NT0 + 4]
        outs = refs[NT0 + 4:]
        dis = dis_ref[...]
        s = jnp.concatenate(
            [s0p_ref[0, c] + s0p_ref[1, c] + g0_refs[c][...]
             for c in range(NT0)], axis=1)
        a = dis * s
        h1 = jnp.maximum(
            jnp.dot(a, W1_ref[...], preferred_element_type=jnp.float32)
            + b1_ref[...], 0.0)
        g1 = dis * jnp.dot(h1, W2_ref[...], preferred_element_type=jnp.float32)
        for c in range(NT1):
            outs[c][...] = g1[:, c * _TC:(c + 1) * _TC]

    return pl.pallas_call(
        body,
        grid=(N // BR,),
        in_specs=[pl.BlockSpec((_NC, NT0, BR, _TC), lambda i: (0, 0, i, 0))]
        + [pl.BlockSpec((BR, _TC), lambda i: (i, 0))] * NT0
        + [
            pl.BlockSpec((BR, 1), lambda i: (i, 0)),
            pl.BlockSpec((F, H1), lambda i: (0, 0)),
            pl.BlockSpec((1, H1), lambda i: (0, 0)),
            pl.BlockSpec((H1, H2), lambda i: (0, 0)),
        ],
        out_specs=[pl.BlockSpec((BR, _TC), lambda i: (i, 0))] * NT1,
        out_shape=[jax.ShapeDtypeStruct((N, _TC), jnp.float32)] * NT1,
    )(s0p, *g0s, dis, W1, b1, W2)


def _conv2_call(s1p, g1s, dis, b2, W3, N, BR):
    NT1 = len(g1s)
    H2 = NT1 * _TC
    H3 = W3.shape[1]
    NT2 = H3 // _TC

    def body(s1p_ref, *refs):
        g1_refs = refs[:NT1]
        dis_ref, b2_ref, W3_ref = refs[NT1:NT1 + 3]
        outs = refs[NT1 + 3:]
        dis = dis_ref[...]
        s = jnp.concatenate(
            [s1p_ref[0, c] + s1p_ref[1, c] + g1_refs[c][...]
             for c in range(NT1)], axis=1)
        h2 = jnp.maximum(dis * s + b2_ref[...], 0.0)
        g2 = dis * jnp.dot(h2, W3_ref[...], preferred_element_type=jnp.float32)
        for c in range(NT2):
            outs[c][...] = g2[:, c * _TC:(c + 1) * _TC]

    return pl.pallas_call(
        body,
        grid=(N // BR,),
        in_specs=[pl.BlockSpec((_NC, NT1, BR, _TC), lambda i: (0, 0, i, 0))]
        + [pl.BlockSpec((BR, _TC), lambda i: (i, 0))] * NT1
        + [
            pl.BlockSpec((BR, 1), lambda i: (i, 0)),
            pl.BlockSpec((1, H2), lambda i: (0, 0)),
            pl.BlockSpec((H2, H3), lambda i: (0, 0)),
        ],
        out_specs=[pl.BlockSpec((BR, _TC), lambda i: (i, 0))] * NT2,
        out_shape=[jax.ShapeDtypeStruct((N, _TC), jnp.float32)] * NT2,
    )(s1p, *g1s, dis, b2, W3)


def _conv3_call(s2p, g2s, dis, b3, Wfc, bfc, N, BR):
    NT2 = len(g2s)
    H3 = NT2 * _TC
    C = Wfc.shape[1]

    def body(s2p_ref, *refs):
        g2_refs = refs[:NT2]
        dis_ref, b3_ref, Wfc_ref, bfc_ref, out_ref = refs[NT2:]
        dis = dis_ref[...]
        s = jnp.concatenate(
            [s2p_ref[0, c] + s2p_ref[1, c] + g2_refs[c][...]
             for c in range(NT2)], axis=1)
        h3 = jnp.maximum(dis * s + b3_ref[...], 0.0)
        out_ref[...] = (
            jnp.dot(h3, Wfc_ref[...], preferred_element_type=jnp.float32)
            + bfc_ref[...])

    return pl.pallas_call(
        body,
        grid=(N // BR,),
        in_specs=[pl.BlockSpec((_NC, NT2, BR, _TC), lambda i: (0, 0, i, 0))]
        + [pl.BlockSpec((BR, _TC), lambda i: (i, 0))] * NT2
        + [
            pl.BlockSpec((BR, 1), lambda i: (i, 0)),
            pl.BlockSpec((1, H3), lambda i: (0, 0)),
            pl.BlockSpec((H3, C), lambda i: (0, 0)),
            pl.BlockSpec((1, C), lambda i: (0, 0)),
        ],
        out_specs=pl.BlockSpec((BR, C), lambda i: (i, 0)),
        out_shape=jax.ShapeDtypeStruct((N, C), jnp.float32),
    )(s2p, *g2s, dis, b3, Wfc, bfc)


def kernel(x, edge_index, W1, b1, W2, b2, W3, b3, Wfc, bfc):
    N, F = x.shape
    E = edge_index.shape[1]
    H2 = W2.shape[1]

    grain = _NW * _K * 8
    EP = ((E + grain - 1) // grain) * grain
    NP = ((N + 1 + 2047) // 2048) * 2048
    BR = _row_block(N)

    src = edge_index[0]
    dst = edge_index[1]
    if EP > E:
        pad = EP - E
        src = jnp.concatenate([src, jnp.zeros((pad,), jnp.int32)])
        dst = jnp.concatenate([dst, jnp.full((pad,), N, jnp.int32)])
    src2 = src.reshape(EP // _K, _K)
    dst2 = dst.reshape(EP // _K, _K)

    degp = _bincount(dst2, NP, EP)
    dis, *g0s = _disg0_call(degp, x, N, BR)

    s0p = _segsum(tuple(g0s), src2, dst2, len(g0s), NP, EP, 6)
    g1s = _conv1_call(s0p, g0s, dis, W1, b1.reshape(1, -1), W2, N, BR)

    s1p = _segsum(tuple(g1s), src2, dst2, len(g1s), NP, EP, 6)
    g2s = _conv2_call(s1p, g1s, dis, b2.reshape(1, -1), W3, N, BR)

    s2p = _segsum(tuple(g2s), src2, dst2, len(g2s), NP, EP, 6)
    out = _conv3_call(s2p, g2s, dis, b3.reshape(1, -1), Wfc,
                      bfc.reshape(1, -1), N, BR)
    return out
